# Initial kernel scaffold; baseline (speedup 1.0000x reference)
import jax
import jax.numpy as jnp
from jax.experimental import pallas as pl


def kernel(x, edge_index, batch, protein_embedding, W_neigh, W_self, b_sage, W_prot, b_prot, W_inter, b_inter, W_out, b_out):
    raise NotImplementedError("write your pallas kernel here")


# speedup vs baseline: 3.2169x; 3.2169x over previous
import functools

import jax
import jax.numpy as jnp
from jax import lax
from jax.experimental import pallas as pl
from jax.experimental.pallas import tpu as pltpu
from jax.experimental.pallas import tpu_sc as plsc

_N = 10000
_E = 320000
_D = 128
_H = 128
_P = 1024
_G = 512
_K = 64
_NCHUNK = _E // _K
_NSUB = 16
_RPT = 624
_REM = _N - _NSUB * _RPT
_BN = 1000
_NB = _N // _BN

_ROWCHUNKS = []
_off = 0
while _off < _RPT:
    _sz = min(_K, _RPT - _off)
    _ROWCHUNKS.append((_off, _sz))
    _off += _sz


def _edge_agg_body(x_hbm, src_hbm, dst_hbm, zrows_hbm, ones_hbm,
                   acc_out, src_v, dst_v, rows_v, ones_v, acc_sh, sem):
    cid = lax.axis_index("c")
    sid = lax.axis_index("s")
    r0 = pl.multiple_of(sid * _RPT, 8)
    w0 = pl.multiple_of(cid * _N + r0, 8)
    pltpu.sync_copy(zrows_hbm, rows_v)
    for off, sz in _ROWCHUNKS:
        pltpu.sync_copy(rows_v.at[pl.ds(0, sz)],
                        acc_sh.at[pl.ds(r0 + off, sz)])

    @pl.when(sid == _NSUB - 1)
    def _():
        pltpu.sync_copy(rows_v.at[pl.ds(0, _REM)],
                        acc_sh.at[pl.ds(_NSUB * _RPT, _REM)])

    pltpu.sync_copy(ones_hbm, ones_v)
    plsc.subcore_barrier()

    niter = (_NCHUNK - 1 - sid) // _NSUB + 1

    @pl.when(cid == 0)
    def _():
        def bodya(j, carry):
            chunk = sid + _NSUB * j
            base = pl.multiple_of(chunk * _K, _K)
            pltpu.sync_copy(src_hbm.at[pl.ds(base, _K)], src_v)
            pltpu.sync_copy(dst_hbm.at[pl.ds(base, _K)], dst_v)
            pltpu.async_copy(x_hbm.at[src_v], rows_v, sem).wait()
            pltpu.sync_copy(rows_v, acc_sh.at[dst_v], add=True)
            return carry

        lax.fori_loop(0, niter, bodya, 0)

    @pl.when(cid == 1)
    def _():
        def bodyb(j, carry):
            chunk = sid + _NSUB * j
            base = pl.multiple_of(chunk * _K, _K)
            pltpu.sync_copy(dst_hbm.at[pl.ds(base, _K)], dst_v)
            pltpu.sync_copy(ones_v, acc_sh.at[dst_v], add=True)
            return carry

        lax.fori_loop(0, niter, bodyb, 0)

    plsc.subcore_barrier()
    for off, sz in _ROWCHUNKS:
        pltpu.sync_copy(acc_sh.at[pl.ds(r0 + off, sz)],
                        rows_v.at[pl.ds(0, sz)])
        pltpu.sync_copy(rows_v.at[pl.ds(0, sz)],
                        acc_out.at[pl.ds(w0 + off, sz)])

    @pl.when(sid == _NSUB - 1)
    def _():
        w1 = pl.multiple_of(cid * _N + _NSUB * _RPT, 8)
        pltpu.sync_copy(acc_sh.at[pl.ds(_NSUB * _RPT, _REM)],
                        rows_v.at[pl.ds(0, _REM)])
        pltpu.sync_copy(rows_v.at[pl.ds(0, _REM)],
                        acc_out.at[pl.ds(w1, _REM)])


@functools.cache
def _edge_agg_kernel():
    return pl.kernel(
        _edge_agg_body,
        mesh=plsc.VectorSubcoreMesh(core_axis_name="c", subcore_axis_name="s"),
        out_type=jax.ShapeDtypeStruct((2 * _N, _D), jnp.float32),
        scratch_types=[
            pltpu.VMEM((_K,), jnp.int32),
            pltpu.VMEM((_K,), jnp.int32),
            pltpu.VMEM((_K, _D), jnp.float32),
            pltpu.VMEM((_K, _D), jnp.float32),
            pltpu.VMEM_SHARED((_N, _D), jnp.float32),
            pltpu.SemaphoreType.DMA,
        ],
    )


def _tc_body(agg_ref, cnt_ref, x_ref, batch_ref,
             wn_ref, ws_ref, bs_ref, pe_ref, wp_ref, bp_ref,
             wi_ref, bi_ref, wo_ref, bo_ref, out_ref, acc_sum, acc_cnt):
    i = pl.program_id(0)

    @pl.when(i == 0)
    def _():
        acc_sum[...] = jnp.zeros_like(acc_sum)
        acc_cnt[...] = jnp.zeros_like(acc_cnt)

    agg = agg_ref[...]
    cnt = cnt_ref[...][:, :1]
    mean_agg = agg / jnp.maximum(cnt, 1.0)
    h = jnp.dot(mean_agg, wn_ref[...], preferred_element_type=jnp.float32)
    h = h + jnp.dot(x_ref[...], ws_ref[...], preferred_element_type=jnp.float32)
    h = jax.nn.relu(h + bs_ref[...])
    b = batch_ref[...].reshape(1, _BN)
    gids = lax.broadcasted_iota(jnp.int32, (_G, _BN), 0)
    onehot = (gids == b).astype(jnp.float32)
    acc_sum[...] += jnp.dot(onehot, h, preferred_element_type=jnp.float32)
    acc_cnt[...] += jnp.sum(onehot, axis=1, keepdims=True)

    @pl.when(i == _NB - 1)
    def _():
        pooled = acc_sum[...] / jnp.maximum(acc_cnt[...][:, :1], 1.0)
        prot = jnp.dot(pe_ref[...], wp_ref[...],
                       preferred_element_type=jnp.float32) + bp_ref[...]
        comb = jnp.concatenate([pooled, prot], axis=1)
        inter = jax.nn.relu(
            jnp.dot(comb, wi_ref[...], preferred_element_type=jnp.float32)
            + bi_ref[...])
        out_ref[...] = jnp.dot(inter, wo_ref[...],
                               preferred_element_type=jnp.float32) + bo_ref[...]


def _tc_fused(acc, x, batch3, W_neigh, W_self, b_sage2,
              pe, W_prot, b_prot2, W_inter, b_inter2, W_out, b_out2):
    return pl.pallas_call(
        _tc_body,
        grid=(_NB,),
        in_specs=[
            pl.BlockSpec((_BN, _D), lambda i: (i, 0)),
            pl.BlockSpec((_BN, _D), lambda i: (i + _NB, 0)),
            pl.BlockSpec((_BN, _D), lambda i: (i, 0)),
            pl.BlockSpec((1, 1, _BN), lambda i: (i, 0, 0)),
            pl.BlockSpec((_D, _H), lambda i: (0, 0)),
            pl.BlockSpec((_D, _H), lambda i: (0, 0)),
            pl.BlockSpec((1, _H), lambda i: (0, 0)),
            pl.BlockSpec((_G, _P), lambda i: (0, 0)),
            pl.BlockSpec((_P, _H), lambda i: (0, 0)),
            pl.BlockSpec((1, _H), lambda i: (0, 0)),
            pl.BlockSpec((2 * _H, _H), lambda i: (0, 0)),
            pl.BlockSpec((1, _H), lambda i: (0, 0)),
            pl.BlockSpec((_H, 1), lambda i: (0, 0)),
            pl.BlockSpec((1, 1), lambda i: (0, 0)),
        ],
        out_specs=pl.BlockSpec((_G, 1), lambda i: (0, 0)),
        out_shape=jax.ShapeDtypeStruct((_G, 1), jnp.float32),
        scratch_shapes=[
            pltpu.VMEM((_G, _H), jnp.float32),
            pltpu.VMEM((_G, _H), jnp.float32),
        ],
        compiler_params=pltpu.CompilerParams(
            dimension_semantics=("arbitrary",)),
    )(acc, acc, x, batch3, W_neigh, W_self,
      b_sage2, pe, W_prot, b_prot2, W_inter, b_inter2, W_out, b_out2)


def kernel(x, edge_index, batch, protein_embedding, W_neigh, W_self, b_sage,
           W_prot, b_prot, W_inter, b_inter, W_out, b_out):
    src = edge_index[0]
    dst = edge_index[1]
    zrows = jnp.zeros((_K, _D), jnp.float32)
    ones = jnp.ones((_K, _D), jnp.float32)
    acc = _edge_agg_kernel()(x, src, dst, zrows, ones)
    batch3 = batch.reshape(_NB, 1, _BN)
    return _tc_fused(acc, x, batch3, W_neigh, W_self,
                     b_sage.reshape(1, _H), protein_embedding, W_prot,
                     b_prot.reshape(1, _H), W_inter, b_inter.reshape(1, _H),
                     W_out, b_out.reshape(1, 1))


# speedup vs baseline: 4.3684x; 1.3580x over previous
import functools

import jax
import jax.numpy as jnp
from jax import lax
from jax.experimental import pallas as pl
from jax.experimental.pallas import tpu as pltpu
from jax.experimental.pallas import tpu_sc as plsc

_N = 10000
_E = 320000
_D = 128
_H = 128
_P = 1024
_G = 512
_K = 80
_NCHUNK = _E // _K
_NSUB = 16
_NIT = _NCHUNK // _NSUB
_RPT = 624
_REM = _N - _NSUB * _RPT
_BN = 1000
_NB = _N // _BN

_ROWCHUNKS = []
_off = 0
while _off < _RPT:
    _sz = min(_K, _RPT - _off)
    _ROWCHUNKS.append((_off, _sz))
    _off += _sz


def _edge_agg_body(x_hbm, src_hbm, dst_hbm, zrows_hbm, ones_hbm,
                   acc_out, src_v, dst_v, rows_v, acc_sh, g0, g1, s0, s1):
    cid = lax.axis_index("c")
    sid = lax.axis_index("s")
    r0 = pl.multiple_of(sid * _RPT, 8)
    w0 = pl.multiple_of(cid * _N + r0, 8)
    pltpu.sync_copy(zrows_hbm, rows_v.at[0])
    for off, sz in _ROWCHUNKS:
        pltpu.sync_copy(rows_v.at[0].at[pl.ds(0, sz)],
                        acc_sh.at[pl.ds(r0 + off, sz)])

    @pl.when(sid == _NSUB - 1)
    def _():
        pltpu.sync_copy(rows_v.at[0].at[pl.ds(0, _REM)],
                        acc_sh.at[pl.ds(_NSUB * _RPT, _REM)])

    plsc.subcore_barrier()

    gsem = (g0, g1)
    ssem = (s0, s1)

    def _base(m):
        return pl.multiple_of((sid + _NSUB * m) * _K, _K)

    @pl.when(cid == 0)
    def _():
        def start(m, b):
            base = _base(m)
            pltpu.sync_copy(src_hbm.at[pl.ds(base, _K)], src_v.at[b])
            pltpu.sync_copy(dst_hbm.at[pl.ds(base, _K)], dst_v.at[b])
            pltpu.async_copy(x_hbm.at[src_v.at[b]], rows_v.at[b], gsem[b])

        def gwait(b):
            pltpu.make_async_copy(x_hbm.at[src_v.at[b]], rows_v.at[b],
                                  gsem[b]).wait()

        def sfire(b):
            pltpu.async_copy(rows_v.at[b], acc_sh.at[dst_v.at[b]], ssem[b],
                             add=True)

        def sdrain(b):
            pltpu.make_async_copy(rows_v.at[b], acc_sh.at[dst_v.at[b]],
                                  ssem[b]).wait()

        start(0, 0)

        def outer(t, carry):
            m0 = 2 * t
            gwait(0)

            @pl.when(t > 0)
            def _():
                sdrain(1)

            start(m0 + 1, 1)
            sfire(0)
            gwait(1)
            sdrain(0)

            @pl.when(m0 + 2 < _NIT)
            def _():
                start(m0 + 2, 0)

            sfire(1)
            return carry

        lax.fori_loop(0, _NIT // 2, outer, 0)
        sdrain(1)

    @pl.when(cid == 1)
    def _():
        pltpu.sync_copy(ones_hbm, rows_v.at[0])

        def dload(m, b):
            pltpu.sync_copy(dst_hbm.at[pl.ds(_base(m), _K)], dst_v.at[b])

        def sfire(b):
            pltpu.async_copy(rows_v.at[0], acc_sh.at[dst_v.at[b]], ssem[b],
                             add=True)

        def sdrain(b):
            pltpu.make_async_copy(rows_v.at[0], acc_sh.at[dst_v.at[b]],
                                  ssem[b]).wait()

        dload(0, 0)

        def outer(t, carry):
            m0 = 2 * t
            sfire(0)
            dload(m0 + 1, 1)
            sdrain(0)
            sfire(1)

            @pl.when(m0 + 2 < _NIT)
            def _():
                dload(m0 + 2, 0)

            sdrain(1)
            return carry

        lax.fori_loop(0, _NIT // 2, outer, 0)

    plsc.subcore_barrier()
    for off, sz in _ROWCHUNKS:
        pltpu.sync_copy(acc_sh.at[pl.ds(r0 + off, sz)],
                        rows_v.at[0].at[pl.ds(0, sz)])
        pltpu.sync_copy(rows_v.at[0].at[pl.ds(0, sz)],
                        acc_out.at[pl.ds(w0 + off, sz)])

    @pl.when(sid == _NSUB - 1)
    def _():
        w1 = pl.multiple_of(cid * _N + _NSUB * _RPT, 8)
        pltpu.sync_copy(acc_sh.at[pl.ds(_NSUB * _RPT, _REM)],
                        rows_v.at[0].at[pl.ds(0, _REM)])
        pltpu.sync_copy(rows_v.at[0].at[pl.ds(0, _REM)],
                        acc_out.at[pl.ds(w1, _REM)])


@functools.cache
def _edge_agg_kernel():
    return pl.kernel(
        _edge_agg_body,
        mesh=plsc.VectorSubcoreMesh(core_axis_name="c", subcore_axis_name="s"),
        out_type=jax.ShapeDtypeStruct((2 * _N, _D), jnp.float32),
        scratch_types=[
            pltpu.VMEM((2, _K), jnp.int32),
            pltpu.VMEM((2, _K), jnp.int32),
            pltpu.VMEM((2, _K, _D), jnp.float32),
            pltpu.VMEM_SHARED((_N, _D), jnp.float32),
            pltpu.SemaphoreType.DMA,
            pltpu.SemaphoreType.DMA,
            pltpu.SemaphoreType.DMA,
            pltpu.SemaphoreType.DMA,
        ],
    )


def _tc_body(agg_ref, cnt_ref, x_ref, batch_ref,
             wn_ref, ws_ref, bs_ref, pe_ref, wp_ref, bp_ref,
             wi_ref, bi_ref, wo_ref, bo_ref, out_ref, acc_sum, acc_cnt):
    i = pl.program_id(0)

    @pl.when(i == 0)
    def _():
        acc_sum[...] = jnp.zeros_like(acc_sum)
        acc_cnt[...] = jnp.zeros_like(acc_cnt)

    agg = agg_ref[...]
    cnt = cnt_ref[...][:, :1]
    mean_agg = agg / jnp.maximum(cnt, 1.0)
    h = jnp.dot(mean_agg, wn_ref[...], preferred_element_type=jnp.float32)
    h = h + jnp.dot(x_ref[...], ws_ref[...], preferred_element_type=jnp.float32)
    h = jax.nn.relu(h + bs_ref[...])
    b = batch_ref[...].reshape(1, _BN)
    gids = lax.broadcasted_iota(jnp.int32, (_G, _BN), 0)
    onehot = (gids == b).astype(jnp.float32)
    acc_sum[...] += jnp.dot(onehot, h, preferred_element_type=jnp.float32)
    acc_cnt[...] += jnp.sum(onehot, axis=1, keepdims=True)

    @pl.when(i == _NB - 1)
    def _():
        pooled = acc_sum[...] / jnp.maximum(acc_cnt[...][:, :1], 1.0)
        prot = jnp.dot(pe_ref[...], wp_ref[...],
                       preferred_element_type=jnp.float32) + bp_ref[...]
        comb = jnp.concatenate([pooled, prot], axis=1)
        inter = jax.nn.relu(
            jnp.dot(comb, wi_ref[...], preferred_element_type=jnp.float32)
            + bi_ref[...])
        out_ref[...] = jnp.dot(inter, wo_ref[...],
                               preferred_element_type=jnp.float32) + bo_ref[...]


def _tc_fused(acc, x, batch3, W_neigh, W_self, b_sage2,
              pe, W_prot, b_prot2, W_inter, b_inter2, W_out, b_out2):
    return pl.pallas_call(
        _tc_body,
        grid=(_NB,),
        in_specs=[
            pl.BlockSpec((_BN, _D), lambda i: (i, 0)),
            pl.BlockSpec((_BN, _D), lambda i: (i + _NB, 0)),
            pl.BlockSpec((_BN, _D), lambda i: (i, 0)),
            pl.BlockSpec((1, 1, _BN), lambda i: (i, 0, 0)),
            pl.BlockSpec((_D, _H), lambda i: (0, 0)),
            pl.BlockSpec((_D, _H), lambda i: (0, 0)),
            pl.BlockSpec((1, _H), lambda i: (0, 0)),
            pl.BlockSpec((_G, _P), lambda i: (0, 0)),
            pl.BlockSpec((_P, _H), lambda i: (0, 0)),
            pl.BlockSpec((1, _H), lambda i: (0, 0)),
            pl.BlockSpec((2 * _H, _H), lambda i: (0, 0)),
            pl.BlockSpec((1, _H), lambda i: (0, 0)),
            pl.BlockSpec((_H, 1), lambda i: (0, 0)),
            pl.BlockSpec((1, 1), lambda i: (0, 0)),
        ],
        out_specs=pl.BlockSpec((_G, 1), lambda i: (0, 0)),
        out_shape=jax.ShapeDtypeStruct((_G, 1), jnp.float32),
        scratch_shapes=[
            pltpu.VMEM((_G, _H), jnp.float32),
            pltpu.VMEM((_G, _H), jnp.float32),
        ],
        compiler_params=pltpu.CompilerParams(
            dimension_semantics=("arbitrary",)),
    )(acc, acc, x, batch3, W_neigh, W_self,
      b_sage2, pe, W_prot, b_prot2, W_inter, b_inter2, W_out, b_out2)


def kernel(x, edge_index, batch, protein_embedding, W_neigh, W_self, b_sage,
           W_prot, b_prot, W_inter, b_inter, W_out, b_out):
    src = edge_index[0]
    dst = edge_index[1]
    zrows = jnp.zeros((_K, _D), jnp.float32)
    ones = jnp.ones((_K, _D), jnp.float32)
    acc = _edge_agg_kernel()(x, src, dst, zrows, ones)
    batch3 = batch.reshape(_NB, 1, _BN)
    return _tc_fused(acc, x, batch3, W_neigh, W_self,
                     b_sage.reshape(1, _H), protein_embedding, W_prot,
                     b_prot.reshape(1, _H), W_inter, b_inter.reshape(1, _H),
                     W_out, b_out.reshape(1, 1))


# speedup vs baseline: 6.6167x; 1.5147x over previous
import functools

import jax
import jax.numpy as jnp
from jax import lax
from jax.experimental import pallas as pl
from jax.experimental.pallas import tpu as pltpu
from jax.experimental.pallas import tpu_sc as plsc

_N = 10000
_E = 320000
_D = 128
_H = 128
_P = 1024
_G = 512
_K = 80
_NCHUNK = _E // _K
_NSUB = 16
_NIT = _NCHUNK // _NSUB
_GSZ = 10
_RPT = 624
_REM = _N - _NSUB * _RPT
_BN = 1000
_NB = _N // _BN

_ROWCHUNKS = []
_off = 0
while _off < _RPT:
    _sz = min(_K, _RPT - _off)
    _ROWCHUNKS.append((_off, _sz))
    _off += _sz


def _edge_agg_body(x_hbm, src_hbm, dst_hbm, zrows_hbm, ones_hbm,
                   acc_out, src_f, dst_v, rows_v, acc_sh,
                   g0, g1, s0, s1, i0, i1):
    cid = lax.axis_index("c")
    sid = lax.axis_index("s")
    r0 = pl.multiple_of(sid * _RPT, 8)
    w0 = pl.multiple_of(cid * _N + r0, 8)
    pltpu.sync_copy(zrows_hbm, rows_v.at[0])
    for off, sz in _ROWCHUNKS:
        pltpu.sync_copy(rows_v.at[0].at[pl.ds(0, sz)],
                        acc_sh.at[pl.ds(r0 + off, sz)])

    @pl.when(sid == _NSUB - 1)
    def _():
        pltpu.sync_copy(rows_v.at[0].at[pl.ds(0, _REM)],
                        acc_sh.at[pl.ds(_NSUB * _RPT, _REM)])

    plsc.subcore_barrier()

    gsem = (g0, g1)
    ssem = (s0, s1)
    isem = (i0, i1)

    def _base(m):
        return pl.multiple_of((sid * _NIT + m) * _K, _K)

    @pl.when(cid == 0)
    def _():
        def gfire(q, b):
            pltpu.async_copy(x_hbm.at[src_f.at[pl.ds(q * _K, _K)]],
                             rows_v.at[b], gsem[b])

        def gwait(q, b):
            pltpu.make_async_copy(x_hbm.at[src_f.at[pl.ds(q * _K, _K)]],
                                  rows_v.at[b], gsem[b]).wait()

        def dload(g, q, b):
            pltpu.async_copy(dst_hbm.at[pl.ds(_base(g * _GSZ + q), _K)],
                             dst_v.at[b], isem[b])

        def idrain(g, q, b):
            pltpu.make_async_copy(dst_hbm.at[pl.ds(_base(g * _GSZ + q), _K)],
                                  dst_v.at[b], isem[b]).wait()

        def sfire(b):
            pltpu.async_copy(rows_v.at[b], acc_sh.at[dst_v.at[b]], ssem[b],
                             add=True)

        def sdrain(b):
            pltpu.make_async_copy(rows_v.at[b], acc_sh.at[dst_v.at[b]],
                                  ssem[b]).wait()

        def group(g, carry):
            gb = pl.multiple_of((sid * _NIT + g * _GSZ) * _K, _K)
            pltpu.sync_copy(src_hbm.at[pl.ds(gb, _GSZ * _K)], src_f)
            dload(g, 0, 0)
            gfire(0, 0)
            for q in range(_GSZ):
                b = q & 1
                gwait(q, b)
                if q >= 1:
                    sdrain(1 - b)
                if q + 1 < _GSZ:
                    gfire(q + 1, 1 - b)
                    dload(g, q + 1, 1 - b)
                idrain(g, q, b)
                sfire(b)
            sdrain((_GSZ - 1) & 1)
            return carry

        lax.fori_loop(0, _NIT // _GSZ, group, 0)

    @pl.when(cid == 1)
    def _():
        pltpu.sync_copy(ones_hbm, rows_v.at[0])

        def dload(m, b):
            pltpu.sync_copy(dst_hbm.at[pl.ds(_base(m), _K)], dst_v.at[b])

        def sfire(b):
            pltpu.async_copy(rows_v.at[0], acc_sh.at[dst_v.at[b]], ssem[b],
                             add=True)

        def sdrain(b):
            pltpu.make_async_copy(rows_v.at[0], acc_sh.at[dst_v.at[b]],
                                  ssem[b]).wait()

        dload(0, 0)

        def outer(t, carry):
            m0 = 2 * t
            sfire(0)
            dload(m0 + 1, 1)
            sdrain(0)
            sfire(1)

            @pl.when(m0 + 2 < _NIT)
            def _():
                dload(m0 + 2, 0)

            sdrain(1)
            return carry

        lax.fori_loop(0, _NIT // 2, outer, 0)

    plsc.subcore_barrier()
    for off, sz in _ROWCHUNKS:
        pltpu.sync_copy(acc_sh.at[pl.ds(r0 + off, sz)],
                        rows_v.at[0].at[pl.ds(0, sz)])
        pltpu.sync_copy(rows_v.at[0].at[pl.ds(0, sz)],
                        acc_out.at[pl.ds(w0 + off, sz)])

    @pl.when(sid == _NSUB - 1)
    def _():
        w1 = pl.multiple_of(cid * _N + _NSUB * _RPT, 8)
        pltpu.sync_copy(acc_sh.at[pl.ds(_NSUB * _RPT, _REM)],
                        rows_v.at[0].at[pl.ds(0, _REM)])
        pltpu.sync_copy(rows_v.at[0].at[pl.ds(0, _REM)],
                        acc_out.at[pl.ds(w1, _REM)])


@functools.cache
def _edge_agg_kernel():
    return pl.kernel(
        _edge_agg_body,
        mesh=plsc.VectorSubcoreMesh(core_axis_name="c", subcore_axis_name="s"),
        out_type=jax.ShapeDtypeStruct((2 * _N, _D), jnp.float32),
        scratch_types=[
            pltpu.VMEM((_GSZ * _K,), jnp.int32),
            pltpu.VMEM((2, _K), jnp.int32),
            pltpu.VMEM((2, _K, _D), jnp.float32),
            pltpu.VMEM_SHARED((_N, _D), jnp.float32),
            pltpu.SemaphoreType.DMA,
            pltpu.SemaphoreType.DMA,
            pltpu.SemaphoreType.DMA,
            pltpu.SemaphoreType.DMA,
            pltpu.SemaphoreType.DMA,
            pltpu.SemaphoreType.DMA,
        ],
    )


def _tc_body(agg_ref, cnt_ref, x_ref, batch_ref,
             wn_ref, ws_ref, bs_ref, pe_ref, wp_ref, bp_ref,
             wi_ref, bi_ref, wo_ref, bo_ref, out_ref, acc_sum, acc_cnt):
    i = pl.program_id(0)

    @pl.when(i == 0)
    def _():
        acc_sum[...] = jnp.zeros_like(acc_sum)
        acc_cnt[...] = jnp.zeros_like(acc_cnt)

    agg = agg_ref[...]
    cnt = cnt_ref[...][:, :1]
    mean_agg = agg / jnp.maximum(cnt, 1.0)
    h = jnp.dot(mean_agg, wn_ref[...], preferred_element_type=jnp.float32)
    h = h + jnp.dot(x_ref[...], ws_ref[...], preferred_element_type=jnp.float32)
    h = jax.nn.relu(h + bs_ref[...])
    b = batch_ref[...].reshape(1, _BN)
    gids = lax.broadcasted_iota(jnp.int32, (_G, _BN), 0)
    onehot = (gids == b).astype(jnp.float32)
    acc_sum[...] += jnp.dot(onehot, h, preferred_element_type=jnp.float32)
    acc_cnt[...] += jnp.sum(onehot, axis=1, keepdims=True)

    @pl.when(i == _NB - 1)
    def _():
        pooled = acc_sum[...] / jnp.maximum(acc_cnt[...][:, :1], 1.0)
        prot = jnp.dot(pe_ref[...], wp_ref[...],
                       preferred_element_type=jnp.float32) + bp_ref[...]
        comb = jnp.concatenate([pooled, prot], axis=1)
        inter = jax.nn.relu(
            jnp.dot(comb, wi_ref[...], preferred_element_type=jnp.float32)
            + bi_ref[...])
        out_ref[...] = jnp.dot(inter, wo_ref[...],
                               preferred_element_type=jnp.float32) + bo_ref[...]


def _tc_fused(acc, x, batch3, W_neigh, W_self, b_sage2,
              pe, W_prot, b_prot2, W_inter, b_inter2, W_out, b_out2):
    return pl.pallas_call(
        _tc_body,
        grid=(_NB,),
        in_specs=[
            pl.BlockSpec((_BN, _D), lambda i: (i, 0)),
            pl.BlockSpec((_BN, _D), lambda i: (i + _NB, 0)),
            pl.BlockSpec((_BN, _D), lambda i: (i, 0)),
            pl.BlockSpec((1, 1, _BN), lambda i: (i, 0, 0)),
            pl.BlockSpec((_D, _H), lambda i: (0, 0)),
            pl.BlockSpec((_D, _H), lambda i: (0, 0)),
            pl.BlockSpec((1, _H), lambda i: (0, 0)),
            pl.BlockSpec((_G, _P), lambda i: (0, 0)),
            pl.BlockSpec((_P, _H), lambda i: (0, 0)),
            pl.BlockSpec((1, _H), lambda i: (0, 0)),
            pl.BlockSpec((2 * _H, _H), lambda i: (0, 0)),
            pl.BlockSpec((1, _H), lambda i: (0, 0)),
            pl.BlockSpec((_H, 1), lambda i: (0, 0)),
            pl.BlockSpec((1, 1), lambda i: (0, 0)),
        ],
        out_specs=pl.BlockSpec((_G, 1), lambda i: (0, 0)),
        out_shape=jax.ShapeDtypeStruct((_G, 1), jnp.float32),
        scratch_shapes=[
            pltpu.VMEM((_G, _H), jnp.float32),
            pltpu.VMEM((_G, _H), jnp.float32),
        ],
        compiler_params=pltpu.CompilerParams(
            dimension_semantics=("arbitrary",)),
    )(acc, acc, x, batch3, W_neigh, W_self,
      b_sage2, pe, W_prot, b_prot2, W_inter, b_inter2, W_out, b_out2)


def kernel(x, edge_index, batch, protein_embedding, W_neigh, W_self, b_sage,
           W_prot, b_prot, W_inter, b_inter, W_out, b_out):
    src = edge_index[0]
    dst = edge_index[1]
    zrows = jnp.zeros((_K, _D), jnp.float32)
    ones = jnp.ones((_K, _D), jnp.float32)
    acc = _edge_agg_kernel()(x, src, dst, zrows, ones)
    batch3 = batch.reshape(_NB, 1, _BN)
    return _tc_fused(acc, x, batch3, W_neigh, W_self,
                     b_sage.reshape(1, _H), protein_embedding, W_prot,
                     b_prot.reshape(1, _H), W_inter, b_inter.reshape(1, _H),
                     W_out, b_out.reshape(1, 1))


# speedup vs baseline: 6.8502x; 1.0353x over previous
import functools

import jax
import jax.numpy as jnp
from jax import lax
from jax.experimental import pallas as pl
from jax.experimental.pallas import tpu as pltpu
from jax.experimental.pallas import tpu_sc as plsc

_N = 10000
_E = 320000
_D = 128
_H = 128
_P = 1024
_G = 512
_K = 80
_NCHUNK = _E // _K
_NSUB = 16
_NIT = _NCHUNK // _NSUB
_GSZ = 10
_RPT = 624
_REM = _N - _NSUB * _RPT
_BN = 1000
_NB = _N // _BN

_ROWCHUNKS = []
_off = 0
while _off < _RPT:
    _sz = min(_K, _RPT - _off)
    _ROWCHUNKS.append((_off, _sz))
    _off += _sz


def _edge_agg_body(x_hbm, src_hbm, dst_hbm, zrows_hbm, ones_hbm,
                   acc_out, src_f, dst_v, rows_v, acc_sh,
                   g0, g1, s0, s1, i0, i1):
    cid = lax.axis_index("c")
    sid = lax.axis_index("s")
    r0 = pl.multiple_of(sid * _RPT, 8)
    w0 = pl.multiple_of(cid * _N + r0, 8)
    pltpu.sync_copy(zrows_hbm, rows_v.at[0])
    for off, sz in _ROWCHUNKS:
        pltpu.sync_copy(rows_v.at[0].at[pl.ds(0, sz)],
                        acc_sh.at[pl.ds(r0 + off, sz)])

    @pl.when(sid == _NSUB - 1)
    def _():
        pltpu.sync_copy(rows_v.at[0].at[pl.ds(0, _REM)],
                        acc_sh.at[pl.ds(_NSUB * _RPT, _REM)])

    plsc.subcore_barrier()

    gsem = (g0, g1)
    ssem = (s0, s1)
    isem = (i0, i1)

    def _base(m):
        return pl.multiple_of((sid * _NIT + m) * _K, _K)

    @pl.when(cid == 0)
    def _():
        def gfire(q, b):
            pltpu.async_copy(x_hbm.at[src_f.at[pl.ds(q * _K, _K)]],
                             rows_v.at[b], gsem[b])

        def gwait(q, b):
            pltpu.make_async_copy(x_hbm.at[src_f.at[pl.ds(q * _K, _K)]],
                                  rows_v.at[b], gsem[b]).wait()

        def dload(g, q, b):
            pltpu.async_copy(dst_hbm.at[pl.ds(_base(g * _GSZ + q), _K)],
                             dst_v.at[b], isem[b])

        def idrain(g, q, b):
            pltpu.make_async_copy(dst_hbm.at[pl.ds(_base(g * _GSZ + q), _K)],
                                  dst_v.at[b], isem[b]).wait()

        def sfire(b):
            pltpu.async_copy(rows_v.at[b], acc_sh.at[dst_v.at[b]], ssem[b],
                             add=True)

        def sdrain(b):
            pltpu.make_async_copy(rows_v.at[b], acc_sh.at[dst_v.at[b]],
                                  ssem[b]).wait()

        def group(g, carry):
            gb = pl.multiple_of((sid * _NIT + g * _GSZ) * _K, _K)
            pltpu.sync_copy(src_hbm.at[pl.ds(gb, _GSZ * _K)], src_f)
            dload(g, 0, 0)
            gfire(0, 0)
            for q in range(_GSZ):
                b = q & 1
                gwait(q, b)
                idrain(g, q, b)
                sfire(b)
                if q == 0:
                    @pl.when(g > 0)
                    def _():
                        sdrain(1 - b)
                else:
                    sdrain(1 - b)
                if q + 1 < _GSZ:
                    gfire(q + 1, 1 - b)
                    dload(g, q + 1, 1 - b)
            return carry

        lax.fori_loop(0, _NIT // _GSZ, group, 0)
        sdrain((_GSZ - 1) & 1)

    @pl.when(cid == 1)
    def _():
        pltpu.sync_copy(ones_hbm, rows_v.at[0])

        def dload(m, b):
            pltpu.sync_copy(dst_hbm.at[pl.ds(_base(m), _K)], dst_v.at[b])

        def sfire(b):
            pltpu.async_copy(rows_v.at[0], acc_sh.at[dst_v.at[b]], ssem[b],
                             add=True)

        def sdrain(b):
            pltpu.make_async_copy(rows_v.at[0], acc_sh.at[dst_v.at[b]],
                                  ssem[b]).wait()

        dload(0, 0)
        dload(1, 1)

        def outer(t, carry):
            m0 = 2 * t
            sfire(0)
            sfire(1)
            sdrain(0)

            @pl.when(m0 + 2 < _NIT)
            def _():
                dload(m0 + 2, 0)

            sdrain(1)

            @pl.when(m0 + 3 < _NIT)
            def _():
                dload(m0 + 3, 1)

            return carry

        lax.fori_loop(0, _NIT // 2, outer, 0)

    plsc.subcore_barrier()
    for off, sz in _ROWCHUNKS:
        pltpu.sync_copy(acc_sh.at[pl.ds(r0 + off, sz)],
                        rows_v.at[0].at[pl.ds(0, sz)])
        pltpu.sync_copy(rows_v.at[0].at[pl.ds(0, sz)],
                        acc_out.at[pl.ds(w0 + off, sz)])

    @pl.when(sid == _NSUB - 1)
    def _():
        w1 = pl.multiple_of(cid * _N + _NSUB * _RPT, 8)
        pltpu.sync_copy(acc_sh.at[pl.ds(_NSUB * _RPT, _REM)],
                        rows_v.at[0].at[pl.ds(0, _REM)])
        pltpu.sync_copy(rows_v.at[0].at[pl.ds(0, _REM)],
                        acc_out.at[pl.ds(w1, _REM)])


@functools.cache
def _edge_agg_kernel():
    return pl.kernel(
        _edge_agg_body,
        mesh=plsc.VectorSubcoreMesh(core_axis_name="c", subcore_axis_name="s"),
        out_type=jax.ShapeDtypeStruct((2 * _N, _D), jnp.float32),
        scratch_types=[
            pltpu.VMEM((_GSZ * _K,), jnp.int32),
            pltpu.VMEM((2, _K), jnp.int32),
            pltpu.VMEM((2, _K, _D), jnp.float32),
            pltpu.VMEM_SHARED((_N, _D), jnp.float32),
            pltpu.SemaphoreType.DMA,
            pltpu.SemaphoreType.DMA,
            pltpu.SemaphoreType.DMA,
            pltpu.SemaphoreType.DMA,
            pltpu.SemaphoreType.DMA,
            pltpu.SemaphoreType.DMA,
        ],
    )


def _tc_body(agg_ref, cnt_ref, x_ref, batch_ref,
             wn_ref, ws_ref, bs_ref, pe_ref, wp_ref, bp_ref,
             wi_ref, bi_ref, wo_ref, bo_ref, out_ref, acc_sum, acc_cnt):
    i = pl.program_id(0)

    @pl.when(i == 0)
    def _():
        acc_sum[...] = jnp.zeros_like(acc_sum)
        acc_cnt[...] = jnp.zeros_like(acc_cnt)

    agg = agg_ref[...]
    cnt = cnt_ref[...][:, :1]
    mean_agg = agg / jnp.maximum(cnt, 1.0)
    h = jnp.dot(mean_agg, wn_ref[...], preferred_element_type=jnp.float32)
    h = h + jnp.dot(x_ref[...], ws_ref[...], preferred_element_type=jnp.float32)
    h = jax.nn.relu(h + bs_ref[...])
    b = batch_ref[...].reshape(1, _BN)
    gids = lax.broadcasted_iota(jnp.int32, (_G, _BN), 0)
    onehot = (gids == b).astype(jnp.float32)
    acc_sum[...] += jnp.dot(onehot, h, preferred_element_type=jnp.float32)
    acc_cnt[...] += jnp.sum(onehot, axis=1, keepdims=True)

    @pl.when(i == _NB - 1)
    def _():
        pooled = acc_sum[...] / jnp.maximum(acc_cnt[...][:, :1], 1.0)
        prot = jnp.dot(pe_ref[...], wp_ref[...],
                       preferred_element_type=jnp.float32) + bp_ref[...]
        comb = jnp.concatenate([pooled, prot], axis=1)
        inter = jax.nn.relu(
            jnp.dot(comb, wi_ref[...], preferred_element_type=jnp.float32)
            + bi_ref[...])
        out_ref[...] = jnp.dot(inter, wo_ref[...],
                               preferred_element_type=jnp.float32) + bo_ref[...]


def _tc_fused(acc, x, batch3, W_neigh, W_self, b_sage2,
              pe, W_prot, b_prot2, W_inter, b_inter2, W_out, b_out2):
    return pl.pallas_call(
        _tc_body,
        grid=(_NB,),
        in_specs=[
            pl.BlockSpec((_BN, _D), lambda i: (i, 0)),
            pl.BlockSpec((_BN, _D), lambda i: (i + _NB, 0)),
            pl.BlockSpec((_BN, _D), lambda i: (i, 0)),
            pl.BlockSpec((1, 1, _BN), lambda i: (i, 0, 0)),
            pl.BlockSpec((_D, _H), lambda i: (0, 0)),
            pl.BlockSpec((_D, _H), lambda i: (0, 0)),
            pl.BlockSpec((1, _H), lambda i: (0, 0)),
            pl.BlockSpec((_G, _P), lambda i: (0, 0)),
            pl.BlockSpec((_P, _H), lambda i: (0, 0)),
            pl.BlockSpec((1, _H), lambda i: (0, 0)),
            pl.BlockSpec((2 * _H, _H), lambda i: (0, 0)),
            pl.BlockSpec((1, _H), lambda i: (0, 0)),
            pl.BlockSpec((_H, 1), lambda i: (0, 0)),
            pl.BlockSpec((1, 1), lambda i: (0, 0)),
        ],
        out_specs=pl.BlockSpec((_G, 1), lambda i: (0, 0)),
        out_shape=jax.ShapeDtypeStruct((_G, 1), jnp.float32),
        scratch_shapes=[
            pltpu.VMEM((_G, _H), jnp.float32),
            pltpu.VMEM((_G, _H), jnp.float32),
        ],
        compiler_params=pltpu.CompilerParams(
            dimension_semantics=("arbitrary",)),
    )(acc, acc, x, batch3, W_neigh, W_self,
      b_sage2, pe, W_prot, b_prot2, W_inter, b_inter2, W_out, b_out2)


def kernel(x, edge_index, batch, protein_embedding, W_neigh, W_self, b_sage,
           W_prot, b_prot, W_inter, b_inter, W_out, b_out):
    src = edge_index[0]
    dst = edge_index[1]
    zrows = jnp.zeros((_K, _D), jnp.float32)
    ones = jnp.ones((_K, _D), jnp.float32)
    acc = _edge_agg_kernel()(x, src, dst, zrows, ones)
    batch3 = batch.reshape(_NB, 1, _BN)
    return _tc_fused(acc, x, batch3, W_neigh, W_self,
                     b_sage.reshape(1, _H), protein_embedding, W_prot,
                     b_prot.reshape(1, _H), W_inter, b_inter.reshape(1, _H),
                     W_out, b_out.reshape(1, 1))
